# baseline (speedup 1.0000x reference)
import functools

import jax
import jax.numpy as jnp
from jax import lax
from jax.experimental import pallas as pl
from jax.experimental.pallas import tpu as pltpu
from jax.experimental.pallas import tpu_sc as plsc

_B = 512
_NSEQ = 64
_AG = 8
_CD = 16
_K = 8192
_BETA = 0.5
_NZ = 7168
_NGATHER = 5120
_CHUNK = 512

_F32 = jnp.float32


def _dot(a, b):
    return jax.lax.dot_general(a, b, (((1,), (0,)), ((), ())),
                               preferred_element_type=_F32)


def _enc2(x, w1, b1, w2, b2):
    h = jnp.maximum(_dot(x, w1) + b1, 0.0)
    return _dot(h, w2) + b2


def _encode_body(past_ref, abs_ref, gt_ref, ep_ref,
                 npw1, npb1, npw2, npb2,
                 apw1, apb1, apw2, apb2,
                 ngw1, ngb1, ngw2, ngb2,
                 sw1, sb1, sw2, sb2,
                 nps_ref, ngs_ref, soc_ref):
    nps = _enc2(past_ref[:], npw1[:], npb1[:], npw2[:], npb2[:])
    aps = _enc2(abs_ref[:], apw1[:], apb1[:], apw2[:], apb2[:])
    ngs = _enc2(gt_ref[:], ngw1[:], ngb1[:], ngw2[:], ngb2[:])
    r = lax.broadcasted_iota(jnp.int32, (_B, _B), 0)
    c = lax.broadcasted_iota(jnp.int32, (_B, _B), 1)
    avg = jnp.where((r // _AG) == (c // _AG), 1.0 / _AG, 0.0).astype(_F32)
    pooled = _dot(avg, aps)
    h = jnp.maximum(_dot(aps, sw1[0:64, :]) + _dot(pooled, sw1[64:128, :])
                    + _dot(ep_ref[:], sw1[128:130, :]) + sb1[:], 0.0)
    soc = _dot(h, sw2[:]) + sb2[:]
    nps_ref[:] = nps
    ngs_ref[:] = ngs
    soc_ref[:] = soc


_STRIP = 512
_NSTRIP = _NZ // _STRIP


def _argmin_body(zt_ref, emb_ref, idx_ref, qs_ref, eaug_ref):
    i = pl.program_id(0)

    @pl.when(i == 0)
    def _():
        e = emb_ref[:]
        en = jnp.sum(e * e, axis=1, keepdims=True)
        eaug_ref[:, :] = jnp.concatenate([-2.0 * e, en], axis=1)
        qs_ref[:, :] = jnp.zeros((1, 1), _F32)

    zb = zt_ref[:]
    zb_h = zb.astype(jnp.bfloat16)

    def step(ci, carry):
        val, bid = carry
        ea = eaug_ref[pl.ds(ci * _CHUNK, _CHUNK), :]
        s = jax.lax.dot_general(ea.astype(jnp.bfloat16), zb_h,
                                (((1,), (0,)), ((), ())),
                                preferred_element_type=_F32)
        for r in range(_CHUNK // 8):
            v = lax.slice(s, (r * 8, 0), (r * 8 + 8, _STRIP))
            upd = v < val
            val = jnp.where(upd, v, val)
            bid = jnp.where(upd, ci * (_CHUNK // 8) + r, bid)
        return val, bid

    val0 = jnp.full((8, _STRIP), jnp.inf, _F32)
    bid0 = jnp.zeros((8, _STRIP), jnp.int32)
    val, bid = lax.fori_loop(0, _K // _CHUNK, step, (val0, bid0))

    fidx = bid * 8 + lax.broadcasted_iota(jnp.int32, (8, _STRIP), 0)
    m = jnp.min(val, axis=0, keepdims=True)
    cand = jnp.where(val == m, fidx, _K)
    idx_ref[0, :, :] = jnp.min(cand, axis=0, keepdims=True)
    zn = jnp.sum(zb * zb, axis=0, keepdims=True) - 1.0
    qs_ref[:, :] += jnp.sum(m + zn, axis=1, keepdims=True)


def _sc_gather(table, idx):
    info = plsc.get_sparse_core_info()
    nw = info.num_cores * info.num_subcores
    b_per_w = _NGATHER // nw
    mesh = plsc.VectorSubcoreMesh(core_axis_name="c", subcore_axis_name="s")

    @functools.partial(
        pl.kernel, mesh=mesh,
        compiler_params=pltpu.CompilerParams(use_tc_tiling_on_sc=False),
        out_type=jax.ShapeDtypeStruct((_NGATHER, _CD), _F32),
        scratch_types=[
            pltpu.VMEM((b_per_w,), jnp.int32),
            pltpu.VMEM((b_per_w, _CD), _F32),
            pltpu.SemaphoreType.DMA,
        ],
    )
    def k(table_hbm, idx_hbm, out_hbm, idx_v, rows_v, sem):
        wid = lax.axis_index("s") * info.num_cores + lax.axis_index("c")
        base = wid * b_per_w
        pltpu.sync_copy(idx_hbm.at[pl.ds(base, b_per_w)], idx_v)
        pltpu.async_copy(table_hbm.at[idx_v], rows_v, sem).wait()
        pltpu.sync_copy(rows_v, out_hbm.at[pl.ds(base, b_per_w)])

    return k(table, idx)


_BF16 = jnp.bfloat16


def _doth(a, b):
    return jax.lax.dot_general(a, b.astype(_BF16), (((1,), (0,)), ((), ())),
                               preferred_element_type=_F32)


def _mlp4(a, b, c, w1, b1, w2, b2, w3, b3, w4, b4):
    h = jnp.maximum(_doth(a, w1[0:64, :]) + _doth(b, w1[64:128, :])
                    + _doth(c, w1[128:224, :]) + b1[:], 0.0)
    h = h.astype(_BF16)
    h = jnp.maximum(_doth(h, w2[:]) + b2[:], 0.0).astype(_BF16)
    h = jnp.maximum(_doth(h, w3[:]) + b3[:], 0.0).astype(_BF16)
    return _doth(h, w4[:]) + b4[:]


def _decode_body(nps_ref, ngs_ref, soc_ref, zqp_ref, zqg_ref, past_ref,
                 rw1, rb1, rw2, rb2,
                 *dec_refs):
    soc = soc_ref[:].astype(_BF16)
    fp = (nps_ref[:] + zqp_ref[:]).astype(_BF16)
    fg = (ngs_ref[:] + zqg_ref[:]).astype(_BF16)

    def run(gi, a):
        g = dec_refs[gi * 8:(gi + 1) * 8]
        return _mlp4(a, soc, fg, *g)

    g1 = run(0, fp)
    x1 = run(1, fp)
    de = _enc2(past_ref[:] - x1, rw1[:], rb1[:], rw2[:], rb2[:])
    de = de.astype(_BF16)
    x2 = run(2, de)
    g2 = run(3, de)
    dec_refs[33][:] = x1 + x2
    dec_refs[32][:] = g1 + g2


def _dec_operands(p):
    return (p['w1'], p['b1'].reshape(1, -1),
            p['w2'], p['b2'].reshape(1, -1),
            p['w3'], p['b3'].reshape(1, -1),
            p['w4'], p['b4'].reshape(1, -1))


def kernel(past, abs_past, seq_start_end, end_pose, future, ground_truth, params):
    del seq_start_end, future
    p = params
    past2 = past.reshape(_B, -1)
    abs2 = abs_past.reshape(_B, -1)
    gt2 = ground_truth.reshape(_B, -1)

    def b2(b):
        return b.reshape(1, -1)

    npe, ape, nge, soc, rpe = p['npe'], p['ape'], p['nge'], p['soc'], p['rpe']
    nps, ngs, socs = pl.pallas_call(
        _encode_body,
        out_shape=(
            jax.ShapeDtypeStruct((_B, 64), _F32),
            jax.ShapeDtypeStruct((_B, 96), _F32),
            jax.ShapeDtypeStruct((_B, 64), _F32),
        ),
    )(past2, abs2, gt2, end_pose,
      npe['w1'], b2(npe['b1']), npe['w2'], b2(npe['b2']),
      ape['w1'], b2(ape['b1']), ape['w2'], b2(ape['b2']),
      nge['w1'], b2(nge['b1']), nge['w2'], b2(nge['b2']),
      soc['w1'], b2(soc['b1']), soc['w2'], b2(soc['b2']))

    z = jnp.concatenate([nps.reshape(-1, _CD), ngs.reshape(-1, _CD),
                         socs.reshape(-1, _CD)], axis=0)
    zt = jnp.concatenate([z, jnp.ones((_NZ, 1), _F32)], axis=1).T

    idx, qsum = pl.pallas_call(
        _argmin_body,
        grid=(_NSTRIP,),
        in_specs=[
            pl.BlockSpec((17, _STRIP), lambda i: (0, i)),
            pl.BlockSpec((_K, _CD), lambda i: (0, 0)),
        ],
        out_specs=(
            pl.BlockSpec((1, 1, _STRIP), lambda i: (i, 0, 0)),
            pl.BlockSpec((1, 1), lambda i: (0, 0)),
        ),
        out_shape=(
            jax.ShapeDtypeStruct((_NSTRIP, 1, _STRIP), jnp.int32),
            jax.ShapeDtypeStruct((1, 1), _F32),
        ),
        scratch_shapes=[pltpu.VMEM((_K, _CD + 1), _F32)],
    )(zt, p['codebook'])

    q_loss = (1.0 + _BETA) * qsum[0, 0] / (_NZ * _CD)

    zq = _sc_gather(p['codebook'], idx.reshape(-1)[:_NGATHER])
    zqp = zq[:2048].reshape(_B, 64)
    zqg = zq[2048:].reshape(_B, 96)

    outs = pl.pallas_call(
        _decode_body,
        out_shape=(
            jax.ShapeDtypeStruct((_B, 2 * 20), _F32),
            jax.ShapeDtypeStruct((_B, 2 * 8), _F32),
        ),
    )(nps, ngs, socs, zqp, zqg, past2,
      rpe['w1'], b2(rpe['b1']), rpe['w2'], b2(rpe['b2']),
      *_dec_operands(p['dec_gt']), *_dec_operands(p['dec_x']),
      *_dec_operands(p['dec_2_x']), *_dec_operands(p['dec_2_gt']))

    gt_out, rec_out = outs
    return (rec_out.reshape(_B, _AG, 2), gt_out.reshape(_B, 20, 2), q_loss)


# speedup vs baseline: 1.2600x; 1.2600x over previous
import functools

import jax
import jax.numpy as jnp
from jax import lax
from jax.experimental import pallas as pl
from jax.experimental.pallas import tpu as pltpu
from jax.experimental.pallas import tpu_sc as plsc

_B = 512
_NSEQ = 64
_AG = 8
_CD = 16
_K = 8192
_BETA = 0.5
_NZ = 7168
_NGATHER = 5120
_CHUNK = 512

_F32 = jnp.float32


def _dot(a, b):
    return jax.lax.dot_general(a, b, (((1,), (0,)), ((), ())),
                               preferred_element_type=_F32)


def _enc2(x, w1, b1, w2, b2):
    h = jnp.maximum(_dot(x, w1) + b1, 0.0)
    return _dot(h, w2) + b2


def _encode_body(past_ref, abs_ref, gt_ref, ep_ref,
                 npw1, npb1, npw2, npb2,
                 apw1, apb1, apw2, apb2,
                 ngw1, ngb1, ngw2, ngb2,
                 sw1, sb1, sw2, sb2,
                 nps_ref, ngs_ref, soc_ref):
    nps = _enc2(past_ref[:], npw1[:], npb1[:], npw2[:], npb2[:])
    aps = _enc2(abs_ref[:], apw1[:], apb1[:], apw2[:], apb2[:])
    ngs = _enc2(gt_ref[:], ngw1[:], ngb1[:], ngw2[:], ngb2[:])
    r = lax.broadcasted_iota(jnp.int32, (_B, _B), 0)
    c = lax.broadcasted_iota(jnp.int32, (_B, _B), 1)
    avg = jnp.where((r // _AG) == (c // _AG), 1.0 / _AG, 0.0).astype(_F32)
    pooled = _dot(avg, aps)
    h = jnp.maximum(_dot(aps, sw1[0:64, :]) + _dot(pooled, sw1[64:128, :])
                    + _dot(ep_ref[:], sw1[128:130, :]) + sb1[:], 0.0)
    soc = _dot(h, sw2[:]) + sb2[:]
    nps_ref[:] = nps
    ngs_ref[:] = ngs
    soc_ref[:] = soc


_STRIP = 512
_NSTRIP = _NZ // _STRIP


def _argmin_body(zt_ref, emb_ref, idx_ref, qs_ref, eaug_ref):
    i = pl.program_id(0)

    @pl.when(i == 0)
    def _():
        e = emb_ref[:]
        en = jnp.sum(e * e, axis=1, keepdims=True)
        eaug_ref[:, :] = jnp.concatenate([-2.0 * e, en], axis=1)
        qs_ref[:, :] = jnp.zeros((1, 1), _F32)

    zb = zt_ref[:]
    zb_h = zb.astype(jnp.bfloat16)

    val = jnp.full((8, _STRIP), jnp.inf, _F32)
    bid = jnp.zeros((8, _STRIP), jnp.int32)
    for ci in range(_K // _CHUNK):
        ea = eaug_ref[ci * _CHUNK:(ci + 1) * _CHUNK, :]
        s = jax.lax.dot_general(ea.astype(jnp.bfloat16), zb_h,
                                (((1,), (0,)), ((), ())),
                                preferred_element_type=_F32)
        for r in range(_CHUNK // 8):
            v = lax.slice(s, (r * 8, 0), (r * 8 + 8, _STRIP))
            upd = v < val
            val = jnp.where(upd, v, val)
            bid = jnp.where(upd, ci * (_CHUNK // 8) + r, bid)

    fidx = bid * 8 + lax.broadcasted_iota(jnp.int32, (8, _STRIP), 0)
    m = jnp.min(val, axis=0, keepdims=True)
    cand = jnp.where(val == m, fidx, _K)
    idx_ref[0, :, :] = jnp.min(cand, axis=0, keepdims=True)
    zn = jnp.sum(zb * zb, axis=0, keepdims=True) - 1.0
    qs_ref[:, :] += jnp.sum(m + zn, axis=1, keepdims=True)


def _sc_gather(table, idx):
    info = plsc.get_sparse_core_info()
    nw = info.num_cores * info.num_subcores
    b_per_w = _NGATHER // nw
    mesh = plsc.VectorSubcoreMesh(core_axis_name="c", subcore_axis_name="s")

    @functools.partial(
        pl.kernel, mesh=mesh,
        compiler_params=pltpu.CompilerParams(use_tc_tiling_on_sc=False),
        out_type=jax.ShapeDtypeStruct((_NGATHER, _CD), _F32),
        scratch_types=[
            pltpu.VMEM((b_per_w,), jnp.int32),
            pltpu.VMEM((b_per_w, _CD), _F32),
            pltpu.SemaphoreType.DMA,
        ],
    )
    def k(table_hbm, idx_hbm, out_hbm, idx_v, rows_v, sem):
        wid = lax.axis_index("s") * info.num_cores + lax.axis_index("c")
        base = wid * b_per_w
        pltpu.sync_copy(idx_hbm.at[pl.ds(base, b_per_w)], idx_v)
        pltpu.async_copy(table_hbm.at[idx_v], rows_v, sem).wait()
        pltpu.sync_copy(rows_v, out_hbm.at[pl.ds(base, b_per_w)])

    return k(table, idx)


_BF16 = jnp.bfloat16


def _doth(a, b):
    return jax.lax.dot_general(a, b.astype(_BF16), (((1,), (0,)), ((), ())),
                               preferred_element_type=_F32)


def _mlp4(a, b, c, w1, b1, w2, b2, w3, b3, w4, b4):
    h = jnp.maximum(_doth(a, w1[0:64, :]) + _doth(b, w1[64:128, :])
                    + _doth(c, w1[128:224, :]) + b1[:], 0.0)
    h = h.astype(_BF16)
    h = jnp.maximum(_doth(h, w2[:]) + b2[:], 0.0).astype(_BF16)
    h = jnp.maximum(_doth(h, w3[:]) + b3[:], 0.0).astype(_BF16)
    return _doth(h, w4[:]) + b4[:]


def _decode_body(nps_ref, ngs_ref, soc_ref, zqp_ref, zqg_ref, past_ref,
                 rw1, rb1, rw2, rb2,
                 *dec_refs):
    soc = soc_ref[:].astype(_BF16)
    fp = (nps_ref[:] + zqp_ref[:]).astype(_BF16)
    fg = (ngs_ref[:] + zqg_ref[:]).astype(_BF16)

    def run(gi, a):
        g = dec_refs[gi * 8:(gi + 1) * 8]
        return _mlp4(a, soc, fg, *g)

    g1 = run(0, fp)
    x1 = run(1, fp)
    de = _enc2(past_ref[:] - x1, rw1[:], rb1[:], rw2[:], rb2[:])
    de = de.astype(_BF16)
    x2 = run(2, de)
    g2 = run(3, de)
    dec_refs[33][:] = x1 + x2
    dec_refs[32][:] = g1 + g2


def _dec_operands(p):
    return (p['w1'], p['b1'].reshape(1, -1),
            p['w2'], p['b2'].reshape(1, -1),
            p['w3'], p['b3'].reshape(1, -1),
            p['w4'], p['b4'].reshape(1, -1))


def kernel(past, abs_past, seq_start_end, end_pose, future, ground_truth, params):
    del seq_start_end, future
    p = params
    past2 = past.reshape(_B, -1)
    abs2 = abs_past.reshape(_B, -1)
    gt2 = ground_truth.reshape(_B, -1)

    def b2(b):
        return b.reshape(1, -1)

    npe, ape, nge, soc, rpe = p['npe'], p['ape'], p['nge'], p['soc'], p['rpe']
    nps, ngs, socs = pl.pallas_call(
        _encode_body,
        out_shape=(
            jax.ShapeDtypeStruct((_B, 64), _F32),
            jax.ShapeDtypeStruct((_B, 96), _F32),
            jax.ShapeDtypeStruct((_B, 64), _F32),
        ),
    )(past2, abs2, gt2, end_pose,
      npe['w1'], b2(npe['b1']), npe['w2'], b2(npe['b2']),
      ape['w1'], b2(ape['b1']), ape['w2'], b2(ape['b2']),
      nge['w1'], b2(nge['b1']), nge['w2'], b2(nge['b2']),
      soc['w1'], b2(soc['b1']), soc['w2'], b2(soc['b2']))

    z = jnp.concatenate([nps.reshape(-1, _CD), ngs.reshape(-1, _CD),
                         socs.reshape(-1, _CD)], axis=0)
    zt = jnp.concatenate([z, jnp.ones((_NZ, 1), _F32)], axis=1).T

    idx, qsum = pl.pallas_call(
        _argmin_body,
        grid=(_NSTRIP,),
        in_specs=[
            pl.BlockSpec((17, _STRIP), lambda i: (0, i)),
            pl.BlockSpec((_K, _CD), lambda i: (0, 0)),
        ],
        out_specs=(
            pl.BlockSpec((1, 1, _STRIP), lambda i: (i, 0, 0)),
            pl.BlockSpec((1, 1), lambda i: (0, 0)),
        ),
        out_shape=(
            jax.ShapeDtypeStruct((_NSTRIP, 1, _STRIP), jnp.int32),
            jax.ShapeDtypeStruct((1, 1), _F32),
        ),
        scratch_shapes=[pltpu.VMEM((_K, _CD + 1), _F32)],
    )(zt, p['codebook'])

    q_loss = (1.0 + _BETA) * qsum[0, 0] / (_NZ * _CD)

    zq = _sc_gather(p['codebook'], idx.reshape(-1)[:_NGATHER])
    zqp = zq[:2048].reshape(_B, 64)
    zqg = zq[2048:].reshape(_B, 96)

    outs = pl.pallas_call(
        _decode_body,
        out_shape=(
            jax.ShapeDtypeStruct((_B, 2 * 20), _F32),
            jax.ShapeDtypeStruct((_B, 2 * 8), _F32),
        ),
    )(nps, ngs, socs, zqp, zqg, past2,
      rpe['w1'], b2(rpe['b1']), rpe['w2'], b2(rpe['b2']),
      *_dec_operands(p['dec_gt']), *_dec_operands(p['dec_x']),
      *_dec_operands(p['dec_2_x']), *_dec_operands(p['dec_2_gt']))

    gt_out, rec_out = outs
    return (rec_out.reshape(_B, _AG, 2), gt_out.reshape(_B, 20, 2), q_loss)


# speedup vs baseline: 1.3624x; 1.0813x over previous
import functools

import jax
import jax.numpy as jnp
from jax import lax
from jax.experimental import pallas as pl
from jax.experimental.pallas import tpu as pltpu
from jax.experimental.pallas import tpu_sc as plsc

_B = 512
_NSEQ = 64
_AG = 8
_CD = 16
_K = 8192
_BETA = 0.5
_NZ = 7168
_NGATHER = 5120
_CHUNK = 512

_F32 = jnp.float32


def _dot(a, b):
    return jax.lax.dot_general(a, b, (((1,), (0,)), ((), ())),
                               preferred_element_type=_F32)


def _enc2(x, w1, b1, w2, b2):
    h = jnp.maximum(_dot(x, w1) + b1, 0.0)
    return _dot(h, w2) + b2


_STRIP = 512
_NSTRIP = _NZ // _STRIP


def _encmin_body(past_ref, abs_ref, gt_ref, ep_ref,
                 npw1, npb1, npw2, npb2,
                 apw1, apb1, apw2, apb2,
                 ngw1, ngb1, ngw2, ngb2,
                 sw1, sb1, sw2, sb2,
                 emb_ref,
                 nps_ref, ngs_ref, soc_ref, idx_ref, qs_ref,
                 zts_ref, eaug_ref):
    i = pl.program_id(0)

    @pl.when(i == 0)
    def _():
        nps = _enc2(past_ref[:], npw1[:], npb1[:], npw2[:], npb2[:])
        aps = _enc2(abs_ref[:], apw1[:], apb1[:], apw2[:], apb2[:])
        ngs = _enc2(gt_ref[:], ngw1[:], ngb1[:], ngw2[:], ngb2[:])
        r = lax.broadcasted_iota(jnp.int32, (_B, _B), 0)
        c = lax.broadcasted_iota(jnp.int32, (_B, _B), 1)
        avg = jnp.where((r // _AG) == (c // _AG), 1.0 / _AG, 0.0).astype(_F32)
        pooled = _dot(avg, aps)
        h = jnp.maximum(_dot(aps, sw1[0:64, :]) + _dot(pooled, sw1[64:128, :])
                        + _dot(ep_ref[:], sw1[128:130, :]) + sb1[:], 0.0)
        soc = _dot(h, sw2[:]) + sb2[:]
        nps_ref[:] = nps
        ngs_ref[:] = ngs
        soc_ref[:] = soc
        npt = jnp.swapaxes(nps, 0, 1)
        ngt = jnp.swapaxes(ngs, 0, 1)
        sct = jnp.swapaxes(soc, 0, 1)
        one = jnp.ones((1, _STRIP), _F32)
        for g in range(4):
            zts_ref[g, 0:16, :] = lax.slice(npt, (16 * g, 0), (16 * g + 16, _B))
            zts_ref[g, 16:17, :] = one
        for g in range(6):
            zts_ref[4 + g, 0:16, :] = lax.slice(ngt, (16 * g, 0), (16 * g + 16, _B))
            zts_ref[4 + g, 16:17, :] = one
        for g in range(4):
            zts_ref[10 + g, 0:16, :] = lax.slice(sct, (16 * g, 0), (16 * g + 16, _B))
            zts_ref[10 + g, 16:17, :] = one
        e = emb_ref[:]
        en = jnp.sum(e * e, axis=1, keepdims=True)
        eaug_ref[:, :] = jnp.concatenate([-2.0 * e, en], axis=1)
        qs_ref[:, :] = jnp.zeros((1, 1), _F32)

    zb = zts_ref[i]
    zb_h = zb.astype(jnp.bfloat16)

    val = jnp.full((8, _STRIP), jnp.inf, _F32)
    bid = jnp.zeros((8, _STRIP), jnp.int32)
    for ci in range(_K // _CHUNK):
        ea = eaug_ref[ci * _CHUNK:(ci + 1) * _CHUNK, :]
        s = jax.lax.dot_general(ea.astype(jnp.bfloat16), zb_h,
                                (((1,), (0,)), ((), ())),
                                preferred_element_type=_F32)
        for r in range(_CHUNK // 8):
            v = lax.slice(s, (r * 8, 0), (r * 8 + 8, _STRIP))
            upd = v < val
            val = jnp.where(upd, v, val)
            bid = jnp.where(upd, ci * (_CHUNK // 8) + r, bid)

    fidx = bid * 8 + lax.broadcasted_iota(jnp.int32, (8, _STRIP), 0)
    m = jnp.min(val, axis=0, keepdims=True)
    cand = jnp.where(val == m, fidx, _K)
    idx_ref[0, :, :] = jnp.min(cand, axis=0, keepdims=True)
    zn = jnp.sum(zb * zb, axis=0, keepdims=True) - 1.0
    qs_ref[:, :] += jnp.sum(m + zn, axis=1, keepdims=True)


def _sc_gather(table, idx):
    info = plsc.get_sparse_core_info()
    nw = info.num_cores * info.num_subcores
    b_per_w = _NGATHER // nw
    mesh = plsc.VectorSubcoreMesh(core_axis_name="c", subcore_axis_name="s")

    @functools.partial(
        pl.kernel, mesh=mesh,
        compiler_params=pltpu.CompilerParams(use_tc_tiling_on_sc=False),
        out_type=jax.ShapeDtypeStruct((_NGATHER, _CD), _F32),
        scratch_types=[
            pltpu.VMEM((b_per_w,), jnp.int32),
            pltpu.VMEM((b_per_w, _CD), _F32),
            pltpu.SemaphoreType.DMA,
        ],
    )
    def k(table_hbm, idx_hbm, out_hbm, idx_v, rows_v, sem):
        wid = lax.axis_index("s") * info.num_cores + lax.axis_index("c")
        base = wid * b_per_w
        pltpu.sync_copy(idx_hbm.at[pl.ds(base, b_per_w)], idx_v)
        pltpu.async_copy(table_hbm.at[idx_v], rows_v, sem).wait()
        pltpu.sync_copy(rows_v, out_hbm.at[pl.ds(base, b_per_w)])

    return k(table, idx)


_BF16 = jnp.bfloat16


def _doth(a, b):
    return jax.lax.dot_general(a, b.astype(_BF16), (((1,), (0,)), ((), ())),
                               preferred_element_type=_F32)


def _mlp4(a, b, c, w1, b1, w2, b2, w3, b3, w4, b4):
    h = jnp.maximum(_doth(a, w1[0:64, :]) + _doth(b, w1[64:128, :])
                    + _doth(c, w1[128:224, :]) + b1[:], 0.0)
    h = h.astype(_BF16)
    h = jnp.maximum(_doth(h, w2[:]) + b2[:], 0.0).astype(_BF16)
    h = jnp.maximum(_doth(h, w3[:]) + b3[:], 0.0).astype(_BF16)
    return _doth(h, w4[:]) + b4[:]


def _decode_body(nps_ref, ngs_ref, soc_ref, zqp_ref, zqg_ref, past_ref,
                 rw1, rb1, rw2, rb2,
                 *dec_refs):
    soc = soc_ref[:].astype(_BF16)
    fp = (nps_ref[:] + zqp_ref[:]).astype(_BF16)
    fg = (ngs_ref[:] + zqg_ref[:]).astype(_BF16)

    def run(gi, a):
        g = dec_refs[gi * 8:(gi + 1) * 8]
        return _mlp4(a, soc, fg, *g)

    g1 = run(0, fp)
    x1 = run(1, fp)
    de = _enc2(past_ref[:] - x1, rw1[:], rb1[:], rw2[:], rb2[:])
    de = de.astype(_BF16)
    x2 = run(2, de)
    g2 = run(3, de)
    dec_refs[33][:] = x1 + x2
    dec_refs[32][:] = g1 + g2


def _dec_operands(p):
    return (p['w1'], p['b1'].reshape(1, -1),
            p['w2'], p['b2'].reshape(1, -1),
            p['w3'], p['b3'].reshape(1, -1),
            p['w4'], p['b4'].reshape(1, -1))


def kernel(past, abs_past, seq_start_end, end_pose, future, ground_truth, params):
    del seq_start_end, future
    p = params
    past2 = past.reshape(_B, -1)
    abs2 = abs_past.reshape(_B, -1)
    gt2 = ground_truth.reshape(_B, -1)

    def b2(b):
        return b.reshape(1, -1)

    npe, ape, nge, soc, rpe = p['npe'], p['ape'], p['nge'], p['soc'], p['rpe']
    ops = (past2, abs2, gt2, end_pose,
           npe['w1'], b2(npe['b1']), npe['w2'], b2(npe['b2']),
           ape['w1'], b2(ape['b1']), ape['w2'], b2(ape['b2']),
           nge['w1'], b2(nge['b1']), nge['w2'], b2(nge['b2']),
           soc['w1'], b2(soc['b1']), soc['w2'], b2(soc['b2']),
           p['codebook'])
    nps, ngs, socs, idx, qsum = pl.pallas_call(
        _encmin_body,
        grid=(_NSTRIP,),
        in_specs=[pl.BlockSpec(o.shape, lambda i, n=o.ndim: (0,) * n)
                  for o in ops],
        out_specs=(
            pl.BlockSpec((_B, 64), lambda i: (0, 0)),
            pl.BlockSpec((_B, 96), lambda i: (0, 0)),
            pl.BlockSpec((_B, 64), lambda i: (0, 0)),
            pl.BlockSpec((1, 1, _STRIP), lambda i: (i, 0, 0)),
            pl.BlockSpec((1, 1), lambda i: (0, 0)),
        ),
        out_shape=(
            jax.ShapeDtypeStruct((_B, 64), _F32),
            jax.ShapeDtypeStruct((_B, 96), _F32),
            jax.ShapeDtypeStruct((_B, 64), _F32),
            jax.ShapeDtypeStruct((_NSTRIP, 1, _STRIP), jnp.int32),
            jax.ShapeDtypeStruct((1, 1), _F32),
        ),
        scratch_shapes=[pltpu.VMEM((_NSTRIP, _CD + 1, _STRIP), _F32),
                        pltpu.VMEM((_K, _CD + 1), _F32)],
    )(*ops)

    q_loss = (1.0 + _BETA) * qsum[0, 0] / (_NZ * _CD)

    zq = _sc_gather(p['codebook'], idx.reshape(-1)[:_NGATHER])
    zqp = zq[:2048].reshape(4, _B, _CD).transpose(1, 0, 2).reshape(_B, 64)
    zqg = zq[2048:].reshape(6, _B, _CD).transpose(1, 0, 2).reshape(_B, 96)

    outs = pl.pallas_call(
        _decode_body,
        out_shape=(
            jax.ShapeDtypeStruct((_B, 2 * 20), _F32),
            jax.ShapeDtypeStruct((_B, 2 * 8), _F32),
        ),
    )(nps, ngs, socs, zqp, zqg, past2,
      rpe['w1'], b2(rpe['b1']), rpe['w2'], b2(rpe['b2']),
      *_dec_operands(p['dec_gt']), *_dec_operands(p['dec_x']),
      *_dec_operands(p['dec_2_x']), *_dec_operands(p['dec_2_gt']))

    gt_out, rec_out = outs
    return (rec_out.reshape(_B, _AG, 2), gt_out.reshape(_B, 20, 2), q_loss)
